# baseline (speedup 1.0000x reference)
import functools

import jax
import jax.numpy as jnp
from jax import lax
from jax.experimental import pallas as pl
from jax.experimental.pallas import tpu as pltpu
from jax.experimental.pallas import tpu_sc as plsc

_NW = 16
_L = 16


def _fifth_root(u):
    um = jnp.maximum(u, 1e-30)
    bits = plsc.bitcast(um, jnp.int32)
    y = plsc.bitcast(bits // 5 + 852282573, jnp.float32)
    for _ in range(4):
        y2 = y * y
        y = 0.2 * (4.0 * y + um / (y2 * y2))
    return jnp.where(u > 0.0, y, 0.0)


def _make_sc_kernel(n_pad, n_pos, n_gt, n_cls):
    chunk = n_pad // _NW
    ppw = n_pos // _NW
    mesh = plsc.VectorSubcoreMesh(
        core_axis_name="c", subcore_axis_name="s", num_cores=1)

    @functools.partial(
        pl.kernel,
        out_type=jax.ShapeDtypeStruct((n_pad,), jnp.float32),
        mesh=mesh,
        compiler_params=pltpu.CompilerParams(
            needs_layout_passes=False, use_tc_tiling_on_sc=False),
        scratch_types=[
            pltpu.VMEM((chunk,), jnp.float32),
            pltpu.VMEM((ppw,), jnp.int32),
            pltpu.VMEM((4 * ppw,), jnp.int32),
            pltpu.VMEM((4 * ppw,), jnp.float32),
            pltpu.VMEM((ppw, n_cls), jnp.float32),
            pltpu.VMEM((ppw * n_cls,), jnp.float32),
            pltpu.VMEM((4 * n_gt,), jnp.float32),
            pltpu.VMEM((n_gt,), jnp.int32),
            pltpu.VMEM((ppw,), jnp.float32),
            pltpu.SemaphoreType.DMA,
            pltpu.SemaphoreType.DMA,
        ],
    )
    def sc_kernel(cls_scores, bbox_flat, gtb_flat, positive_inds, gt_labels,
                  out,
                  zero_v, idx_v, bidx_v, boxc_v, cls_v, clsf_v, gtb_v, gtl_v,
                  val_v, sem0, sem1):
        w = lax.axis_index("s")

        def zbody(i, carry):
            zero_v[pl.ds(i * _L, _L)] = jnp.zeros((_L,), jnp.float32)
            return carry
        lax.fori_loop(0, chunk // _L, zbody, 0)
        pltpu.sync_copy(zero_v, out.at[pl.ds(w * chunk, chunk)])
        plsc.subcore_barrier()

        pltpu.sync_copy(positive_inds.at[pl.ds(w * ppw, ppw)], idx_v)
        for c in range(4):
            for k in range(ppw // _L):
                pi = idx_v[pl.ds(k * _L, _L)]
                bidx_v[pl.ds(c * ppw + k * _L, _L)] = pi * 4 + c
        cp_box = pltpu.async_copy(bbox_flat.at[bidx_v], boxc_v, sem0)
        cp_cls = pltpu.async_copy(cls_scores.at[idx_v], cls_v, sem1)
        pltpu.sync_copy(gtb_flat, gtb_v)
        pltpu.sync_copy(gt_labels, gtl_v)
        cp_box.wait()
        cp_cls.wait()
        for p in range(ppw):
            for k in range(n_cls // _L):
                clsf_v[pl.ds(p * n_cls + k * _L, _L)] = \
                    cls_v[p, pl.ds(k * _L, _L)]

        lanes = lax.iota(jnp.int32, _L)
        zeros_i = jnp.zeros((_L,), jnp.int32)
        for pb in range(ppw // _L):
            ax1 = boxc_v[pl.ds(0 * ppw + pb * _L, _L)]
            ay1 = boxc_v[pl.ds(1 * ppw + pb * _L, _L)]
            ax2 = boxc_v[pl.ds(2 * ppw + pb * _L, _L)]
            ay2 = boxc_v[pl.ds(3 * ppw + pb * _L, _L)]
            area1 = (ax2 - ax1) * (ay2 - ay1)
            cls_base = (lanes + pb * _L) * n_cls

            def gbody(g, m):
                lab = plsc.load_gather(gtl_v, [zeros_i + g])
                gx1 = plsc.load_gather(gtb_v, [zeros_i + g * 4])
                gy1 = plsc.load_gather(gtb_v, [zeros_i + (g * 4 + 1)])
                gx2 = plsc.load_gather(gtb_v, [zeros_i + (g * 4 + 2)])
                gy2 = plsc.load_gather(gtb_v, [zeros_i + (g * 4 + 3)])
                sraw = plsc.load_gather(clsf_v, [cls_base + lab])
                s = 1.0 / (1.0 + jnp.exp(-sraw))
                area2 = (gx2 - gx1) * (gy2 - gy1)
                iw = jnp.maximum(
                    jnp.minimum(ax2, gx2) - jnp.maximum(ax1, gx1), 0.0)
                ih = jnp.maximum(
                    jnp.minimum(ay2, gy2) - jnp.maximum(ay1, gy1), 0.0)
                inter = iw * ih
                union = jnp.maximum(area1 + area2 - inter, 1e-6)
                iou = inter / union
                iou2 = iou * iou
                return jnp.maximum(m, s * (iou2 * iou2))

            m = lax.fori_loop(0, n_gt, gbody, jnp.zeros((_L,), jnp.float32))
            val_v[pl.ds(pb * _L, _L)] = _fifth_root(m)

        pltpu.async_copy(val_v, out.at[idx_v], sem0).wait()

    return sc_kernel


def kernel(bboxes, cls_scores, bbox_preds, gt_bboxes, bbox_levels,
           positive_inds, gt_labels):
    del bboxes, bbox_levels
    n, n_cls = cls_scores.shape
    n_pos = positive_inds.shape[0]
    n_gt = gt_bboxes.shape[0]
    chunk = -(-n // (_NW * _L)) * _L
    sc_kernel = _make_sc_kernel(chunk * _NW, n_pos, n_gt, n_cls)
    out = sc_kernel(cls_scores,
                    bbox_preds[:, :4].reshape(-1),
                    gt_bboxes[:, :4].reshape(-1),
                    positive_inds.astype(jnp.int32),
                    gt_labels.astype(jnp.int32))
    return out[:n]
